# Initial kernel scaffold; baseline (speedup 1.0000x reference)
import functools

import jax
import jax.numpy as jnp
from jax import lax
from jax.experimental import pallas as pl
from jax.experimental.pallas import tpu as pltpu
from jax.experimental.pallas import tpu_sc as plsc

N_NODES = 10000
N_EDGES = 320000
D_FEAT = 128
D_HID = 16
N_CLASSES = 10

NC, NS = 2, 16
NW = NC * NS
CHUNK = 128
K_CHUNKS = -(-N_EDGES // (NW * CHUNK))
E_PAD = NW * CHUNK * K_CHUNKS

ACC_ROWS_PER_SUBCORE = 626
ACC_ROWS = NS * ACC_ROWS_PER_SUBCORE

_mesh = plsc.VectorSubcoreMesh(core_axis_name="c", subcore_axis_name="s")


def _edge_pass_kernel(g_hbm, src_hbm, dst_hbm, zeros_hbm, out_hbm,
                      acc_sh, src_v, dst_v, rows_v, sem):
    c = lax.axis_index("c")
    s = lax.axis_index("s")
    w = c * NS + s
    row0 = s * ACC_ROWS_PER_SUBCORE
    pltpu.sync_copy(zeros_hbm.at[pl.ds(row0, ACC_ROWS_PER_SUBCORE)],
                    acc_sh.at[pl.ds(row0, ACC_ROWS_PER_SUBCORE)])
    pltpu.sync_copy(src_hbm.at[w], src_v)
    pltpu.sync_copy(dst_hbm.at[w], dst_v)
    plsc.subcore_barrier()

    @pl.loop(0, K_CHUNKS)
    def _(j):
        pltpu.async_copy(g_hbm.at[src_v.at[j]], rows_v, sem).wait()
        pltpu.sync_copy(rows_v, acc_sh.at[dst_v.at[j]], add=True)

    plsc.subcore_barrier()
    pltpu.sync_copy(acc_sh.at[pl.ds(row0, ACC_ROWS_PER_SUBCORE)],
                    out_hbm.at[c, pl.ds(row0, ACC_ROWS_PER_SUBCORE)])


_edge_pass = pl.kernel(
    _edge_pass_kernel,
    out_type=jax.ShapeDtypeStruct((NC, ACC_ROWS, D_HID), jnp.float32),
    mesh=_mesh,
    scratch_types=[
        pltpu.VMEM_SHARED((ACC_ROWS, D_HID), jnp.float32),
        pltpu.VMEM((K_CHUNKS, CHUNK), jnp.int32),
        pltpu.VMEM((K_CHUNKS, CHUNK), jnp.int32),
        pltpu.VMEM((CHUNK, D_HID), jnp.float32),
        pltpu.SemaphoreType.DMA,
    ],
)


def _deg_pass_kernel(dst_hbm, zeros_hbm, ones_hbm, out_hbm,
                     acc_sh, dst_v, ones_v, sem):
    c = lax.axis_index("c")
    s = lax.axis_index("s")
    w = c * NS + s
    row0 = s * ACC_ROWS_PER_SUBCORE
    pltpu.sync_copy(zeros_hbm.at[pl.ds(row0, ACC_ROWS_PER_SUBCORE)],
                    acc_sh.at[pl.ds(row0, ACC_ROWS_PER_SUBCORE)])
    pltpu.sync_copy(dst_hbm.at[w], dst_v)
    pltpu.sync_copy(ones_hbm, ones_v)
    plsc.subcore_barrier()

    @pl.loop(0, K_CHUNKS)
    def _(j):
        pltpu.sync_copy(ones_v, acc_sh.at[dst_v.at[j]], add=True)

    plsc.subcore_barrier()
    pltpu.sync_copy(acc_sh.at[pl.ds(row0, ACC_ROWS_PER_SUBCORE)],
                    out_hbm.at[c, pl.ds(row0, ACC_ROWS_PER_SUBCORE)])


_deg_pass = pl.kernel(
    _deg_pass_kernel,
    out_type=jax.ShapeDtypeStruct((NC, ACC_ROWS, D_HID), jnp.float32),
    mesh=_mesh,
    scratch_types=[
        pltpu.VMEM_SHARED((ACC_ROWS, D_HID), jnp.float32),
        pltpu.VMEM((K_CHUNKS, CHUNK), jnp.int32),
        pltpu.VMEM((CHUNK, D_HID), jnp.float32),
        pltpu.SemaphoreType.DMA,
    ],
)


def _mm1_body(x_ref, w_ref, o_ref):
    o_ref[...] = jnp.dot(x_ref[...], w_ref[...],
                         preferred_element_type=jnp.float32,
                         precision=lax.Precision.HIGHEST)


def _prep_body(degp_ref, h_ref, dinv_ref, g_ref):
    deg = degp_ref[0, :N_NODES, :] + degp_ref[1, :N_NODES, :] + 1.0
    dinv = lax.rsqrt(deg)
    dinv_ref[...] = dinv
    g_ref[...] = dinv * h_ref[...]


def _mid_body(accp_ref, g_ref, dinv_ref, b1_ref, w2_ref, g2_ref):
    agg = accp_ref[0, :N_NODES, :] + accp_ref[1, :N_NODES, :] + g_ref[...]
    h = jnp.maximum(dinv_ref[...] * agg + b1_ref[...], 0.0)
    h2 = jnp.dot(h, w2_ref[...], preferred_element_type=jnp.float32,
                 precision=lax.Precision.HIGHEST)
    g2_ref[...] = dinv_ref[...] * h2


def _final_body(accp_ref, g_ref, dinv_ref, b2_ref, o_ref):
    agg = accp_ref[0, :N_NODES, :] + accp_ref[1, :N_NODES, :] + g_ref[...]
    res = dinv_ref[...] * agg + b2_ref[...]
    o_ref[...] = res[:, :N_CLASSES]


_f32 = jnp.float32


def kernel(x, edge_index, W1, b1, W2, b2):
    ei = edge_index.astype(jnp.int32)
    src = jnp.concatenate(
        [ei[0], jnp.zeros((E_PAD - N_EDGES,), jnp.int32)]).reshape(NW, K_CHUNKS, CHUNK)
    dst = jnp.concatenate(
        [ei[1], jnp.full((E_PAD - N_EDGES,), N_NODES, jnp.int32)]).reshape(NW, K_CHUNKS, CHUNK)
    zeros = jnp.zeros((ACC_ROWS, D_HID), _f32)
    ones = jnp.ones((CHUNK, D_HID), _f32)
    W2p = jnp.pad(W2, ((0, 0), (0, D_HID - N_CLASSES)))
    b1r = b1.reshape(1, D_HID)
    b2r = jnp.pad(b2, (0, D_HID - N_CLASSES)).reshape(1, D_HID)

    degp = _deg_pass(dst, zeros, ones)
    h1 = pl.pallas_call(
        _mm1_body,
        out_shape=jax.ShapeDtypeStruct((N_NODES, D_HID), _f32),
    )(x, W1)
    dinv, g1 = pl.pallas_call(
        _prep_body,
        out_shape=(jax.ShapeDtypeStruct((N_NODES, D_HID), _f32),
                   jax.ShapeDtypeStruct((N_NODES, D_HID), _f32)),
    )(degp, h1)
    acc1 = _edge_pass(g1, src, dst, zeros)
    g2 = pl.pallas_call(
        _mid_body,
        out_shape=jax.ShapeDtypeStruct((N_NODES, D_HID), _f32),
    )(acc1, g1, dinv, b1r, W2p)
    acc2 = _edge_pass(g2, src, dst, zeros)
    out = pl.pallas_call(
        _final_body,
        out_shape=jax.ShapeDtypeStruct((N_NODES, N_CLASSES), _f32),
    )(acc2, g2, dinv, b2r)
    return out


# speedup vs baseline: 33.7782x; 33.7782x over previous
import functools

import jax
import jax.numpy as jnp
from jax import lax
from jax.experimental import pallas as pl
from jax.experimental.pallas import tpu as pltpu
from jax.experimental.pallas import tpu_sc as plsc

N_NODES = 10000
N_EDGES = 320000
D_FEAT = 128
D_HID = 16
N_CLASSES = 10

NC, NS = 2, 16
NW = NC * NS
CHUNK = 128
K_CHUNKS = -(-N_EDGES // (NW * CHUNK))
E_PAD = NW * CHUNK * K_CHUNKS

ACC_ROWS_PER_SUBCORE = 632
ACC_ROWS = NS * ACC_ROWS_PER_SUBCORE

_mesh = plsc.VectorSubcoreMesh(core_axis_name="c", subcore_axis_name="s")


def _edge_pass_kernel(g_hbm, src_hbm, dst_hbm, zeros_hbm, out_hbm,
                      acc_sh, src_v, dst_v, rows_v, sem):
    c = lax.axis_index("c")
    s = lax.axis_index("s")
    w = c * NS + s
    row0 = s * ACC_ROWS_PER_SUBCORE
    pltpu.sync_copy(zeros_hbm.at[pl.ds(row0, ACC_ROWS_PER_SUBCORE)],
                    acc_sh.at[pl.ds(row0, ACC_ROWS_PER_SUBCORE)])
    pltpu.sync_copy(src_hbm.at[w], src_v)
    pltpu.sync_copy(dst_hbm.at[w], dst_v)
    plsc.subcore_barrier()

    @pl.loop(0, K_CHUNKS)
    def _(j):
        pltpu.async_copy(g_hbm.at[src_v.at[j]], rows_v, sem).wait()
        pltpu.sync_copy(rows_v, acc_sh.at[dst_v.at[j]], add=True)

    plsc.subcore_barrier()
    pltpu.sync_copy(acc_sh.at[pl.ds(row0, ACC_ROWS_PER_SUBCORE)],
                    out_hbm.at[c, pl.ds(row0, ACC_ROWS_PER_SUBCORE)])


_edge_pass = pl.kernel(
    _edge_pass_kernel,
    out_type=jax.ShapeDtypeStruct((NC, ACC_ROWS, D_HID), jnp.float32),
    mesh=_mesh,
    scratch_types=[
        pltpu.VMEM_SHARED((ACC_ROWS, D_HID), jnp.float32),
        pltpu.VMEM((K_CHUNKS, CHUNK), jnp.int32),
        pltpu.VMEM((K_CHUNKS, CHUNK), jnp.int32),
        pltpu.VMEM((CHUNK, D_HID), jnp.float32),
        pltpu.SemaphoreType.DMA,
    ],
    compiler_params=pltpu.CompilerParams(use_tc_tiling_on_sc=False),
)


def _deg_pass_kernel(dst_hbm, zeros_hbm, ones_hbm, out_hbm,
                     acc_sh, dst_v, ones_v, sem):
    c = lax.axis_index("c")
    s = lax.axis_index("s")
    w = c * NS + s
    row0 = s * ACC_ROWS_PER_SUBCORE
    pltpu.sync_copy(zeros_hbm.at[pl.ds(row0, ACC_ROWS_PER_SUBCORE)],
                    acc_sh.at[pl.ds(row0, ACC_ROWS_PER_SUBCORE)])
    pltpu.sync_copy(dst_hbm.at[w], dst_v)
    pltpu.sync_copy(ones_hbm, ones_v)
    plsc.subcore_barrier()

    @pl.loop(0, K_CHUNKS)
    def _(j):
        pltpu.sync_copy(ones_v, acc_sh.at[dst_v.at[j]], add=True)

    plsc.subcore_barrier()
    pltpu.sync_copy(acc_sh.at[pl.ds(row0, ACC_ROWS_PER_SUBCORE)],
                    out_hbm.at[c, pl.ds(row0, ACC_ROWS_PER_SUBCORE)])


_deg_pass = pl.kernel(
    _deg_pass_kernel,
    out_type=jax.ShapeDtypeStruct((NC, ACC_ROWS, D_HID), jnp.float32),
    mesh=_mesh,
    scratch_types=[
        pltpu.VMEM_SHARED((ACC_ROWS, D_HID), jnp.float32),
        pltpu.VMEM((K_CHUNKS, CHUNK), jnp.int32),
        pltpu.VMEM((CHUNK, D_HID), jnp.float32),
        pltpu.SemaphoreType.DMA,
    ],
    compiler_params=pltpu.CompilerParams(use_tc_tiling_on_sc=False),
)


def _mm1_body(x_ref, w_ref, o_ref):
    o_ref[...] = jnp.dot(x_ref[...], w_ref[...],
                         preferred_element_type=jnp.float32,
                         precision=lax.Precision.HIGHEST)


def _prep_body(degp_ref, h_ref, dinv_ref, g_ref):
    deg = degp_ref[0, :N_NODES, :] + degp_ref[1, :N_NODES, :] + 1.0
    dinv = lax.rsqrt(deg)
    dinv_ref[...] = dinv
    g_ref[...] = dinv * h_ref[...]


def _mid_body(accp_ref, g_ref, dinv_ref, b1_ref, w2_ref, g2_ref):
    agg = accp_ref[0, :N_NODES, :] + accp_ref[1, :N_NODES, :] + g_ref[...]
    h = jnp.maximum(dinv_ref[...] * agg + b1_ref[...], 0.0)
    h2 = jnp.dot(h, w2_ref[...], preferred_element_type=jnp.float32,
                 precision=lax.Precision.HIGHEST)
    g2_ref[...] = dinv_ref[...] * h2


def _final_body(accp_ref, g_ref, dinv_ref, b2_ref, o_ref):
    agg = accp_ref[0, :N_NODES, :] + accp_ref[1, :N_NODES, :] + g_ref[...]
    res = dinv_ref[...] * agg + b2_ref[...]
    o_ref[...] = res[:, :N_CLASSES]


_f32 = jnp.float32


def kernel(x, edge_index, W1, b1, W2, b2):
    ei = edge_index.astype(jnp.int32)
    src = jnp.concatenate(
        [ei[0], jnp.zeros((E_PAD - N_EDGES,), jnp.int32)]).reshape(NW, K_CHUNKS, CHUNK)
    dst = jnp.concatenate(
        [ei[1], jnp.full((E_PAD - N_EDGES,), N_NODES, jnp.int32)]).reshape(NW, K_CHUNKS, CHUNK)
    zeros = jnp.zeros((ACC_ROWS, D_HID), _f32)
    ones = jnp.ones((CHUNK, D_HID), _f32)
    W2p = jnp.pad(W2, ((0, 0), (0, D_HID - N_CLASSES)))
    b1r = b1.reshape(1, D_HID)
    b2r = jnp.pad(b2, (0, D_HID - N_CLASSES)).reshape(1, D_HID)

    degp = _deg_pass(dst, zeros, ones)
    h1 = pl.pallas_call(
        _mm1_body,
        out_shape=jax.ShapeDtypeStruct((N_NODES, D_HID), _f32),
    )(x, W1)
    dinv, g1 = pl.pallas_call(
        _prep_body,
        out_shape=(jax.ShapeDtypeStruct((N_NODES, D_HID), _f32),
                   jax.ShapeDtypeStruct((N_NODES, D_HID), _f32)),
    )(degp, h1)
    acc1 = _edge_pass(g1, src, dst, zeros)
    g2 = pl.pallas_call(
        _mid_body,
        out_shape=jax.ShapeDtypeStruct((N_NODES, D_HID), _f32),
    )(acc1, g1, dinv, b1r, W2p)
    acc2 = _edge_pass(g2, src, dst, zeros)
    out = pl.pallas_call(
        _final_body,
        out_shape=jax.ShapeDtypeStruct((N_NODES, N_CLASSES), _f32),
    )(acc2, g2, dinv, b2r)
    return out


# speedup vs baseline: 36.1073x; 1.0690x over previous
import jax
import jax.numpy as jnp
from jax import lax
from jax.experimental import pallas as pl
from jax.experimental.pallas import tpu as pltpu
from jax.experimental.pallas import tpu_sc as plsc

N_NODES = 10000
N_EDGES = 320000
D_FEAT = 128
D_HID = 16
N_CLASSES = 10

NC, NS = 2, 16
NW = NC * NS
CHUNK = 128
NBUF = 4
K_CHUNKS = 80
E_PAD = NW * CHUNK * K_CHUNKS
K_MAIN = K_CHUNKS // NBUF - 1

ACC_ROWS_PER_SUBCORE = 632
ACC_ROWS = NS * ACC_ROWS_PER_SUBCORE

_mesh = plsc.VectorSubcoreMesh(core_axis_name="c", subcore_axis_name="s")
_f32 = jnp.float32


def _edge_pass_kernel(g_hbm, src_hbm, dst_hbm, zeros_hbm, out_hbm,
                      acc_sh, src_v, dst_v, r0, r1, r2, r3, s0, s1, s2, s3):
    rows = (r0, r1, r2, r3)
    sems = (s0, s1, s2, s3)
    c = lax.axis_index("c")
    s = lax.axis_index("s")
    w = c * NS + s
    row0 = s * ACC_ROWS_PER_SUBCORE
    pltpu.sync_copy(zeros_hbm.at[pl.ds(row0, ACC_ROWS_PER_SUBCORE)],
                    acc_sh.at[pl.ds(row0, ACC_ROWS_PER_SUBCORE)])
    pltpu.sync_copy(src_hbm.at[w], src_v)
    pltpu.sync_copy(dst_hbm.at[w], dst_v)
    plsc.subcore_barrier()

    def gather_start(j, b):
        pltpu.async_copy(g_hbm.at[src_v.at[j]], rows[b], sems[b])

    def gather_wait(j, b):
        pltpu.make_async_copy(g_hbm.at[src_v.at[j]], rows[b], sems[b]).wait()

    def scatter_add(j, b):
        pltpu.sync_copy(rows[b], acc_sh.at[dst_v.at[j]], add=True)

    for b in range(NBUF - 1):
        gather_start(b, b)

    @pl.loop(0, K_MAIN)
    def _(g):
        base = g * NBUF
        for b in range(NBUF):
            j = base + b
            gather_wait(j, b)
            gather_start(j + NBUF - 1, (b + NBUF - 1) % NBUF)
            scatter_add(j, b)

    tail = K_MAIN * NBUF
    gather_wait(tail, 0)
    gather_start(tail + NBUF - 1, NBUF - 1)
    scatter_add(tail, 0)
    for b in range(1, NBUF):
        gather_wait(tail + b, b)
        scatter_add(tail + b, b)

    plsc.subcore_barrier()
    pltpu.sync_copy(acc_sh.at[pl.ds(row0, ACC_ROWS_PER_SUBCORE)],
                    out_hbm.at[c, pl.ds(row0, ACC_ROWS_PER_SUBCORE)])


_edge_pass = pl.kernel(
    _edge_pass_kernel,
    out_type=jax.ShapeDtypeStruct((NC, ACC_ROWS, D_HID), _f32),
    mesh=_mesh,
    scratch_types=[
        pltpu.VMEM_SHARED((ACC_ROWS, D_HID), _f32),
        pltpu.VMEM((K_CHUNKS, CHUNK), jnp.int32),
        pltpu.VMEM((K_CHUNKS, CHUNK), jnp.int32),
        pltpu.VMEM((CHUNK, D_HID), _f32),
        pltpu.VMEM((CHUNK, D_HID), _f32),
        pltpu.VMEM((CHUNK, D_HID), _f32),
        pltpu.VMEM((CHUNK, D_HID), _f32),
        pltpu.SemaphoreType.DMA,
        pltpu.SemaphoreType.DMA,
        pltpu.SemaphoreType.DMA,
        pltpu.SemaphoreType.DMA,
    ],
    compiler_params=pltpu.CompilerParams(use_tc_tiling_on_sc=False),
)


def _deg_pass_kernel(dst_hbm, zeros_hbm, ones_hbm, out_hbm,
                     acc_sh, dst_v, ones_v, sem):
    c = lax.axis_index("c")
    s = lax.axis_index("s")
    w = c * NS + s
    row0 = s * ACC_ROWS_PER_SUBCORE
    pltpu.sync_copy(zeros_hbm.at[pl.ds(row0, ACC_ROWS_PER_SUBCORE)],
                    acc_sh.at[pl.ds(row0, ACC_ROWS_PER_SUBCORE)])
    pltpu.sync_copy(dst_hbm.at[w], dst_v)
    pltpu.sync_copy(ones_hbm, ones_v)
    plsc.subcore_barrier()

    @pl.loop(0, K_CHUNKS)
    def _(j):
        pltpu.sync_copy(ones_v, acc_sh.at[dst_v.at[j]], add=True)

    plsc.subcore_barrier()
    pltpu.sync_copy(acc_sh.at[pl.ds(row0, ACC_ROWS_PER_SUBCORE)],
                    out_hbm.at[c, pl.ds(row0, ACC_ROWS_PER_SUBCORE)])


_deg_pass = pl.kernel(
    _deg_pass_kernel,
    out_type=jax.ShapeDtypeStruct((NC, ACC_ROWS, D_HID), _f32),
    mesh=_mesh,
    scratch_types=[
        pltpu.VMEM_SHARED((ACC_ROWS, D_HID), _f32),
        pltpu.VMEM((K_CHUNKS, CHUNK), jnp.int32),
        pltpu.VMEM((CHUNK, D_HID), _f32),
        pltpu.SemaphoreType.DMA,
    ],
    compiler_params=pltpu.CompilerParams(use_tc_tiling_on_sc=False),
)


def _prep_body(degp_ref, x_ref, w1_ref, dinv_ref, g_ref):
    h1 = jnp.dot(x_ref[...], w1_ref[...], preferred_element_type=_f32,
                 precision=lax.Precision.HIGHEST)
    deg = degp_ref[0, :N_NODES, :] + degp_ref[1, :N_NODES, :] + 1.0
    dinv = lax.rsqrt(deg)
    dinv_ref[...] = dinv
    g_ref[...] = dinv * h1


def _mid_body(accp_ref, g_ref, dinv_ref, b1_ref, w2_ref, g2_ref):
    agg = accp_ref[0, :N_NODES, :] + accp_ref[1, :N_NODES, :] + g_ref[...]
    h = jnp.maximum(dinv_ref[...] * agg + b1_ref[...], 0.0)
    h2 = jnp.dot(h, w2_ref[...], preferred_element_type=_f32,
                 precision=lax.Precision.HIGHEST)
    g2_ref[...] = dinv_ref[...] * h2


def _final_body(accp_ref, g_ref, dinv_ref, b2_ref, o_ref):
    agg = accp_ref[0, :N_NODES, :] + accp_ref[1, :N_NODES, :] + g_ref[...]
    res = dinv_ref[...] * agg + b2_ref[...]
    o_ref[...] = res[:, :N_CLASSES]


def kernel(x, edge_index, W1, b1, W2, b2):
    ei = edge_index.astype(jnp.int32)
    src = jnp.concatenate(
        [ei[0], jnp.zeros((E_PAD - N_EDGES,), jnp.int32)]).reshape(NW, K_CHUNKS, CHUNK)
    dst = jnp.concatenate(
        [ei[1], jnp.full((E_PAD - N_EDGES,), N_NODES, jnp.int32)]).reshape(NW, K_CHUNKS, CHUNK)
    zeros = jnp.zeros((ACC_ROWS, D_HID), _f32)
    ones = jnp.ones((CHUNK, D_HID), _f32)
    W2p = jnp.pad(W2, ((0, 0), (0, D_HID - N_CLASSES)))
    b1r = b1.reshape(1, D_HID)
    b2r = jnp.pad(b2, (0, D_HID - N_CLASSES)).reshape(1, D_HID)

    degp = _deg_pass(dst, zeros, ones)
    dinv, g1 = pl.pallas_call(
        _prep_body,
        out_shape=(jax.ShapeDtypeStruct((N_NODES, D_HID), _f32),
                   jax.ShapeDtypeStruct((N_NODES, D_HID), _f32)),
    )(degp, x, W1)
    acc1 = _edge_pass(g1, src, dst, zeros)
    g2 = pl.pallas_call(
        _mid_body,
        out_shape=jax.ShapeDtypeStruct((N_NODES, D_HID), _f32),
    )(acc1, g1, dinv, b1r, W2p)
    acc2 = _edge_pass(g2, src, dst, zeros)
    out = pl.pallas_call(
        _final_body,
        out_shape=jax.ShapeDtypeStruct((N_NODES, N_CLASSES), _f32),
    )(acc2, g2, dinv, b2r)
    return out


# speedup vs baseline: 54.5237x; 1.5100x over previous
import jax
import jax.numpy as jnp
from jax import lax
from jax.experimental import pallas as pl
from jax.experimental.pallas import tpu as pltpu
from jax.experimental.pallas import tpu_sc as plsc

N_NODES = 10000
N_EDGES = 320000
D_FEAT = 128
D_HID = 16
N_CLASSES = 10

NC, NS = 2, 16
NW = NC * NS
CHUNK = 128
NBUF = 4
K_CHUNKS = 80
E_PAD = NW * CHUNK * K_CHUNKS
K_MAIN = K_CHUNKS // NBUF - 1

ACC_ROWS_PER_SUBCORE = 632
ACC_ROWS = NS * ACC_ROWS_PER_SUBCORE

_mesh = plsc.VectorSubcoreMesh(core_axis_name="c", subcore_axis_name="s")
_f32 = jnp.float32


def _edge_pass_kernel(g_hbm, src_hbm, dst_hbm, zeros_hbm, out_hbm,
                      acc_sh, g_sh, src_v, dst_v, r0, r1, r2, r3,
                      s0, s1, s2, s3):
    rows = (r0, r1, r2, r3)
    sems = (s0, s1, s2, s3)
    c = lax.axis_index("c")
    s = lax.axis_index("s")
    w = c * NS + s
    row0 = s * ACC_ROWS_PER_SUBCORE
    pltpu.sync_copy(zeros_hbm.at[pl.ds(row0, ACC_ROWS_PER_SUBCORE)],
                    acc_sh.at[pl.ds(row0, ACC_ROWS_PER_SUBCORE)])
    pltpu.sync_copy(g_hbm.at[pl.ds(row0, ACC_ROWS_PER_SUBCORE)],
                    g_sh.at[pl.ds(row0, ACC_ROWS_PER_SUBCORE)])
    pltpu.sync_copy(src_hbm.at[w], src_v)
    pltpu.sync_copy(dst_hbm.at[w], dst_v)
    plsc.subcore_barrier()

    def gather_start(j, b):
        pltpu.async_copy(g_sh.at[src_v.at[j]], rows[b], sems[b])

    def gather_wait(j, b):
        pltpu.make_async_copy(g_sh.at[src_v.at[j]], rows[b], sems[b]).wait()

    def scatter_add(j, b):
        pltpu.sync_copy(rows[b], acc_sh.at[dst_v.at[j]], add=True)

    for b in range(NBUF - 1):
        gather_start(b, b)

    @pl.loop(0, K_MAIN)
    def _(g):
        base = g * NBUF
        for b in range(NBUF):
            j = base + b
            gather_wait(j, b)
            gather_start(j + NBUF - 1, (b + NBUF - 1) % NBUF)
            scatter_add(j, b)

    tail = K_MAIN * NBUF
    gather_wait(tail, 0)
    gather_start(tail + NBUF - 1, NBUF - 1)
    scatter_add(tail, 0)
    for b in range(1, NBUF):
        gather_wait(tail + b, b)
        scatter_add(tail + b, b)

    plsc.subcore_barrier()
    pltpu.sync_copy(acc_sh.at[pl.ds(row0, ACC_ROWS_PER_SUBCORE)],
                    out_hbm.at[c, pl.ds(row0, ACC_ROWS_PER_SUBCORE)])


_edge_pass = pl.kernel(
    _edge_pass_kernel,
    out_type=jax.ShapeDtypeStruct((NC, ACC_ROWS, D_HID), _f32),
    mesh=_mesh,
    scratch_types=[
        pltpu.VMEM_SHARED((ACC_ROWS, D_HID), _f32),
        pltpu.VMEM_SHARED((ACC_ROWS, D_HID), _f32),
        pltpu.VMEM((K_CHUNKS, CHUNK), jnp.int32),
        pltpu.VMEM((K_CHUNKS, CHUNK), jnp.int32),
        pltpu.VMEM((CHUNK, D_HID), _f32),
        pltpu.VMEM((CHUNK, D_HID), _f32),
        pltpu.VMEM((CHUNK, D_HID), _f32),
        pltpu.VMEM((CHUNK, D_HID), _f32),
        pltpu.SemaphoreType.DMA,
        pltpu.SemaphoreType.DMA,
        pltpu.SemaphoreType.DMA,
        pltpu.SemaphoreType.DMA,
    ],
    compiler_params=pltpu.CompilerParams(use_tc_tiling_on_sc=False),
)


def _deg_pass_kernel(dst_hbm, zeros_hbm, ones_hbm, out_hbm,
                     acc_sh, dst_v, ones_v, sem):
    c = lax.axis_index("c")
    s = lax.axis_index("s")
    w = c * NS + s
    row0 = s * ACC_ROWS_PER_SUBCORE
    pltpu.sync_copy(zeros_hbm.at[pl.ds(row0, ACC_ROWS_PER_SUBCORE)],
                    acc_sh.at[pl.ds(row0, ACC_ROWS_PER_SUBCORE)])
    pltpu.sync_copy(dst_hbm.at[w], dst_v)
    pltpu.sync_copy(ones_hbm, ones_v)
    plsc.subcore_barrier()

    @pl.loop(0, K_CHUNKS)
    def _(j):
        pltpu.sync_copy(ones_v, acc_sh.at[dst_v.at[j]], add=True)

    plsc.subcore_barrier()
    pltpu.sync_copy(acc_sh.at[pl.ds(row0, ACC_ROWS_PER_SUBCORE)],
                    out_hbm.at[c, pl.ds(row0, ACC_ROWS_PER_SUBCORE)])


_deg_pass = pl.kernel(
    _deg_pass_kernel,
    out_type=jax.ShapeDtypeStruct((NC, ACC_ROWS, D_HID), _f32),
    mesh=_mesh,
    scratch_types=[
        pltpu.VMEM_SHARED((ACC_ROWS, D_HID), _f32),
        pltpu.VMEM((K_CHUNKS, CHUNK), jnp.int32),
        pltpu.VMEM((CHUNK, D_HID), _f32),
        pltpu.SemaphoreType.DMA,
    ],
    compiler_params=pltpu.CompilerParams(use_tc_tiling_on_sc=False),
)


def _prep_body(degp_ref, x_ref, w1_ref, dinv_ref, g_ref):
    h1 = jnp.dot(x_ref[...], w1_ref[...], preferred_element_type=_f32,
                 precision=lax.Precision.HIGHEST)
    deg = degp_ref[0, :N_NODES, :] + degp_ref[1, :N_NODES, :] + 1.0
    dinv = lax.rsqrt(deg)
    dinv_ref[...] = dinv
    g_ref[0:N_NODES, :] = dinv * h1
    g_ref[N_NODES:ACC_ROWS, :] = jnp.zeros((ACC_ROWS - N_NODES, D_HID), _f32)


def _mid_body(accp_ref, g_ref, dinv_ref, b1_ref, w2_ref, g2_ref):
    agg = (accp_ref[0, :N_NODES, :] + accp_ref[1, :N_NODES, :]
           + g_ref[0:N_NODES, :])
    h = jnp.maximum(dinv_ref[...] * agg + b1_ref[...], 0.0)
    h2 = jnp.dot(h, w2_ref[...], preferred_element_type=_f32,
                 precision=lax.Precision.HIGHEST)
    g2_ref[0:N_NODES, :] = dinv_ref[...] * h2
    g2_ref[N_NODES:ACC_ROWS, :] = jnp.zeros((ACC_ROWS - N_NODES, D_HID), _f32)


def _final_body(accp_ref, g_ref, dinv_ref, b2_ref, o_ref):
    agg = (accp_ref[0, :N_NODES, :] + accp_ref[1, :N_NODES, :]
           + g_ref[0:N_NODES, :])
    res = dinv_ref[...] * agg + b2_ref[...]
    o_ref[...] = res[:, :N_CLASSES]


def kernel(x, edge_index, W1, b1, W2, b2):
    ei = edge_index.astype(jnp.int32)
    src = jnp.concatenate(
        [ei[0], jnp.zeros((E_PAD - N_EDGES,), jnp.int32)]).reshape(NW, K_CHUNKS, CHUNK)
    dst = jnp.concatenate(
        [ei[1], jnp.full((E_PAD - N_EDGES,), N_NODES, jnp.int32)]).reshape(NW, K_CHUNKS, CHUNK)
    zeros = jnp.zeros((ACC_ROWS, D_HID), _f32)
    ones = jnp.ones((CHUNK, D_HID), _f32)
    W2p = jnp.pad(W2, ((0, 0), (0, D_HID - N_CLASSES)))
    b1r = b1.reshape(1, D_HID)
    b2r = jnp.pad(b2, (0, D_HID - N_CLASSES)).reshape(1, D_HID)

    degp = _deg_pass(dst, zeros, ones)
    dinv, g1 = pl.pallas_call(
        _prep_body,
        out_shape=(jax.ShapeDtypeStruct((N_NODES, D_HID), _f32),
                   jax.ShapeDtypeStruct((ACC_ROWS, D_HID), _f32)),
    )(degp, x, W1)
    acc1 = _edge_pass(g1, src, dst, zeros)
    g2 = pl.pallas_call(
        _mid_body,
        out_shape=jax.ShapeDtypeStruct((ACC_ROWS, D_HID), _f32),
    )(acc1, g1, dinv, b1r, W2p)
    acc2 = _edge_pass(g2, src, dst, zeros)
    out = pl.pallas_call(
        _final_body,
        out_shape=jax.ShapeDtypeStruct((N_NODES, N_CLASSES), _f32),
    )(acc2, g2, dinv, b2r)
    return out


# speedup vs baseline: 81.6382x; 1.4973x over previous
import jax
import jax.numpy as jnp
from jax import lax
from jax.experimental import pallas as pl
from jax.experimental.pallas import tpu as pltpu
from jax.experimental.pallas import tpu_sc as plsc

N_NODES = 10000
N_EDGES = 320000
D_FEAT = 128
D_HID = 16
N_CLASSES = 10

NC, NS = 2, 16
NW = NC * NS
CHUNK = 128
NBUF = 4
K_CHUNKS = 80
E_PAD = NW * CHUNK * K_CHUNKS
K_MAIN = K_CHUNKS // NBUF - 1

ACC_ROWS_PER_SUBCORE = 632
ACC_ROWS = NS * ACC_ROWS_PER_SUBCORE
FROWS = ACC_ROWS * D_HID // 128
FROWS_REAL = N_NODES * D_HID // 128

_mesh = plsc.VectorSubcoreMesh(core_axis_name="c", subcore_axis_name="s")
_f32 = jnp.float32


def _edge_pass_kernel(g_hbm, src_hbm, dst_hbm, zeros_hbm, out_hbm,
                      acc_sh, g_sh, src_v, dst_v, r0, r1, r2, r3,
                      s0, s1, s2, s3):
    rows = (r0, r1, r2, r3)
    sems = (s0, s1, s2, s3)
    c = lax.axis_index("c")
    s = lax.axis_index("s")
    w = c * NS + s
    row0 = s * ACC_ROWS_PER_SUBCORE
    pltpu.sync_copy(zeros_hbm.at[pl.ds(row0, ACC_ROWS_PER_SUBCORE)],
                    acc_sh.at[pl.ds(row0, ACC_ROWS_PER_SUBCORE)])
    pltpu.sync_copy(g_hbm.at[pl.ds(row0, ACC_ROWS_PER_SUBCORE)],
                    g_sh.at[pl.ds(row0, ACC_ROWS_PER_SUBCORE)])
    pltpu.sync_copy(src_hbm.at[w], src_v)
    pltpu.sync_copy(dst_hbm.at[w], dst_v)
    plsc.subcore_barrier()

    def gather_start(j, b):
        pltpu.async_copy(g_sh.at[src_v.at[j]], rows[b], sems[b])

    def gather_wait(j, b):
        pltpu.make_async_copy(g_sh.at[src_v.at[j]], rows[b], sems[b]).wait()

    def scatter_add(j, b):
        pltpu.sync_copy(rows[b], acc_sh.at[dst_v.at[j]], add=True)

    for b in range(NBUF - 1):
        gather_start(b, b)

    @pl.loop(0, K_MAIN)
    def _(g):
        base = g * NBUF
        for b in range(NBUF):
            j = base + b
            gather_wait(j, b)
            gather_start(j + NBUF - 1, (b + NBUF - 1) % NBUF)
            scatter_add(j, b)

    tail = K_MAIN * NBUF
    gather_wait(tail, 0)
    gather_start(tail + NBUF - 1, NBUF - 1)
    scatter_add(tail, 0)
    for b in range(1, NBUF):
        gather_wait(tail + b, b)
        scatter_add(tail + b, b)

    plsc.subcore_barrier()
    pltpu.sync_copy(acc_sh.at[pl.ds(row0, ACC_ROWS_PER_SUBCORE)],
                    out_hbm.at[c, pl.ds(row0, ACC_ROWS_PER_SUBCORE)])


_edge_pass = pl.kernel(
    _edge_pass_kernel,
    out_type=jax.ShapeDtypeStruct((NC, ACC_ROWS, D_HID), _f32),
    mesh=_mesh,
    scratch_types=[
        pltpu.VMEM_SHARED((ACC_ROWS, D_HID), _f32),
        pltpu.VMEM_SHARED((ACC_ROWS, D_HID), _f32),
        pltpu.VMEM((K_CHUNKS, CHUNK), jnp.int32),
        pltpu.VMEM((K_CHUNKS, CHUNK), jnp.int32),
        pltpu.VMEM((CHUNK, D_HID), _f32),
        pltpu.VMEM((CHUNK, D_HID), _f32),
        pltpu.VMEM((CHUNK, D_HID), _f32),
        pltpu.VMEM((CHUNK, D_HID), _f32),
        pltpu.SemaphoreType.DMA,
        pltpu.SemaphoreType.DMA,
        pltpu.SemaphoreType.DMA,
        pltpu.SemaphoreType.DMA,
    ],
    compiler_params=pltpu.CompilerParams(use_tc_tiling_on_sc=False),
)


def _deg_pass_kernel(dst_hbm, zeros_hbm, ones_hbm, out_hbm,
                     acc_sh, dst_v, ones_v, sem):
    c = lax.axis_index("c")
    s = lax.axis_index("s")
    w = c * NS + s
    row0 = s * ACC_ROWS_PER_SUBCORE
    pltpu.sync_copy(zeros_hbm.at[pl.ds(row0, ACC_ROWS_PER_SUBCORE)],
                    acc_sh.at[pl.ds(row0, ACC_ROWS_PER_SUBCORE)])
    pltpu.sync_copy(dst_hbm.at[w], dst_v)
    pltpu.sync_copy(ones_hbm, ones_v)
    plsc.subcore_barrier()

    @pl.loop(0, K_CHUNKS)
    def _(j):
        pltpu.sync_copy(ones_v, acc_sh.at[dst_v.at[j]], add=True)

    plsc.subcore_barrier()
    pltpu.sync_copy(acc_sh.at[pl.ds(row0, ACC_ROWS_PER_SUBCORE)],
                    out_hbm.at[c, pl.ds(row0, ACC_ROWS_PER_SUBCORE)])


_deg_pass = pl.kernel(
    _deg_pass_kernel,
    out_type=jax.ShapeDtypeStruct((NC, ACC_ROWS, D_HID), _f32),
    mesh=_mesh,
    scratch_types=[
        pltpu.VMEM_SHARED((ACC_ROWS, D_HID), _f32),
        pltpu.VMEM((K_CHUNKS, CHUNK), jnp.int32),
        pltpu.VMEM((CHUNK, D_HID), _f32),
        pltpu.SemaphoreType.DMA,
    ],
    compiler_params=pltpu.CompilerParams(use_tc_tiling_on_sc=False),
)


def _mm1_body(x_ref, w1_ref, h_ref):
    h_ref[...] = jnp.dot(x_ref[...], w1_ref[...], preferred_element_type=_f32,
                         precision=lax.Precision.HIGHEST)


def _prep2_body(degpf_ref, h1f_ref, dinvf_ref, g1f_ref):
    deg = degpf_ref[0] + degpf_ref[1] + 1.0
    dinvf = lax.rsqrt(deg)
    dinvf_ref[...] = dinvf
    g1f_ref[...] = jnp.concatenate(
        [dinvf[:FROWS_REAL] * h1f_ref[...],
         jnp.zeros((FROWS - FROWS_REAL, 128), _f32)], axis=0)


def _mid_body(accpf_ref, g1f_ref, dinvf_ref, b1f_ref, w2bd_ref, g2f_ref):
    agg = accpf_ref[0] + accpf_ref[1] + g1f_ref[...]
    h = jnp.maximum(dinvf_ref[...] * agg + b1f_ref[...], 0.0)
    h2 = jnp.dot(h, w2bd_ref[...], preferred_element_type=_f32,
                 precision=lax.Precision.HIGHEST)
    g2f_ref[...] = dinvf_ref[...] * h2


def _final_body(accpf_ref, g2f_ref, dinvf_ref, b2f_ref, of_ref):
    agg = accpf_ref[0, :FROWS_REAL] + accpf_ref[1, :FROWS_REAL] \
        + g2f_ref[0:FROWS_REAL]
    of_ref[...] = dinvf_ref[0:FROWS_REAL] * agg + b2f_ref[...]


def kernel(x, edge_index, W1, b1, W2, b2):
    flat = edge_index.astype(jnp.int32).reshape(2 * N_EDGES)
    src = jnp.concatenate(
        [flat[:N_EDGES],
         jnp.zeros((E_PAD - N_EDGES,), jnp.int32)]).reshape(NW, K_CHUNKS, CHUNK)
    dst = jnp.concatenate(
        [flat[N_EDGES:],
         jnp.full((E_PAD - N_EDGES,), N_NODES, jnp.int32)]).reshape(NW, K_CHUNKS, CHUNK)
    zeros = jnp.zeros((ACC_ROWS, D_HID), _f32)
    ones = jnp.ones((CHUNK, D_HID), _f32)
    W2p = jnp.pad(W2, ((0, 0), (0, D_HID - N_CLASSES)))
    w2bd = jnp.kron(jnp.eye(8, dtype=_f32), W2p)
    b1f = jnp.tile(b1, 8).reshape(1, 128)
    b2f = jnp.tile(jnp.pad(b2, (0, D_HID - N_CLASSES)), 8).reshape(1, 128)

    degp = _deg_pass(dst, zeros, ones)
    degpf = degp.reshape(NC, FROWS, 128)
    h1 = pl.pallas_call(
        _mm1_body,
        out_shape=jax.ShapeDtypeStruct((N_NODES, D_HID), _f32),
    )(x, W1)
    h1f = h1.reshape(FROWS_REAL, 128)
    dinvf, g1f = pl.pallas_call(
        _prep2_body,
        out_shape=(jax.ShapeDtypeStruct((FROWS, 128), _f32),
                   jax.ShapeDtypeStruct((FROWS, 128), _f32)),
    )(degpf, h1f)
    acc1 = _edge_pass(g1f.reshape(ACC_ROWS, D_HID), src, dst, zeros)
    g2f = pl.pallas_call(
        _mid_body,
        out_shape=jax.ShapeDtypeStruct((FROWS, 128), _f32),
    )(acc1.reshape(NC, FROWS, 128), g1f, dinvf, b1f, w2bd)
    acc2 = _edge_pass(g2f.reshape(ACC_ROWS, D_HID), src, dst, zeros)
    resf = pl.pallas_call(
        _final_body,
        out_shape=jax.ShapeDtypeStruct((FROWS_REAL, 128), _f32),
    )(acc2.reshape(NC, FROWS, 128), g2f, dinvf, b2f)
    return resf.reshape(N_NODES, D_HID)[:, :N_CLASSES]


# speedup vs baseline: 83.2485x; 1.0197x over previous
import jax
import jax.numpy as jnp
from jax import lax
from jax.experimental import pallas as pl
from jax.experimental.pallas import tpu as pltpu
from jax.experimental.pallas import tpu_sc as plsc

N_NODES = 10000
N_EDGES = 320000
D_FEAT = 128
D_HID = 16
N_CLASSES = 10

NC, NS = 2, 16
NW = NC * NS
CHUNK = 128
NBUF = 8
GAHEAD = 4
K_CHUNKS = 80
E_PAD = NW * CHUNK * K_CHUNKS

ACC_ROWS_PER_SUBCORE = 632
ACC_ROWS = NS * ACC_ROWS_PER_SUBCORE
FROWS = ACC_ROWS * D_HID // 128
FROWS_REAL = N_NODES * D_HID // 128
XF_COLS = 8 * D_FEAT

_mesh = plsc.VectorSubcoreMesh(core_axis_name="c", subcore_axis_name="s")
_f32 = jnp.float32


def _edge_pass_kernel(g_hbm, src_hbm, dst_hbm, zeros_hbm, out_hbm,
                      acc_sh, g_sh, src_v, dst_v,
                      r0, r1, r2, r3, r4, r5, r6, r7,
                      g0, g1, g2, g3, g4, g5, g6, g7,
                      t0, t1, t2, t3, t4, t5, t6, t7):
    rows = (r0, r1, r2, r3, r4, r5, r6, r7)
    gsems = (g0, g1, g2, g3, g4, g5, g6, g7)
    ssems = (t0, t1, t2, t3, t4, t5, t6, t7)
    c = lax.axis_index("c")
    s = lax.axis_index("s")
    w = c * NS + s
    row0 = s * ACC_ROWS_PER_SUBCORE
    pltpu.sync_copy(zeros_hbm.at[pl.ds(row0, ACC_ROWS_PER_SUBCORE)],
                    acc_sh.at[pl.ds(row0, ACC_ROWS_PER_SUBCORE)])
    pltpu.sync_copy(g_hbm.at[pl.ds(row0, ACC_ROWS_PER_SUBCORE)],
                    g_sh.at[pl.ds(row0, ACC_ROWS_PER_SUBCORE)])
    pltpu.sync_copy(src_hbm.at[w], src_v)
    pltpu.sync_copy(dst_hbm.at[w], dst_v)
    plsc.subcore_barrier()

    def gather_start(j, b):
        pltpu.async_copy(g_sh.at[src_v.at[j]], rows[b], gsems[b])

    def gather_wait(j, b):
        pltpu.make_async_copy(g_sh.at[src_v.at[j]], rows[b], gsems[b]).wait()

    def scatter_start(j, b):
        pltpu.async_copy(rows[b], acc_sh.at[dst_v.at[j]], ssems[b], add=True)

    def scatter_wait(j, b):
        pltpu.make_async_copy(rows[b], acc_sh.at[dst_v.at[j]],
                              ssems[b]).wait()

    for b in range(GAHEAD):
        gather_start(b, b)
    for b in range(NBUF):
        j = b
        gather_wait(j, b)
        scatter_start(j, b)
        if j >= GAHEAD:
            scatter_wait(j - GAHEAD, (j - GAHEAD) % NBUF)
        gather_start(j + GAHEAD, (j + GAHEAD) % NBUF)

    @pl.loop(1, K_CHUNKS // NBUF - 1)
    def _(g):
        base = g * NBUF
        for b in range(NBUF):
            j = base + b
            gather_wait(j, b)
            scatter_start(j, b)
            scatter_wait(j - GAHEAD, (b - GAHEAD) % NBUF)
            gather_start(j + GAHEAD, (b + GAHEAD) % NBUF)

    tail = K_CHUNKS - NBUF
    for b in range(NBUF):
        j = tail + b
        gather_wait(j, b)
        scatter_start(j, b)
        scatter_wait(j - GAHEAD, (j - GAHEAD) % NBUF)
        if j + GAHEAD < K_CHUNKS:
            gather_start(j + GAHEAD, (j + GAHEAD) % NBUF)
    for b in range(GAHEAD):
        j = K_CHUNKS - GAHEAD + b
        scatter_wait(j, j % NBUF)

    plsc.subcore_barrier()
    pltpu.sync_copy(acc_sh.at[pl.ds(row0, ACC_ROWS_PER_SUBCORE)],
                    out_hbm.at[c, pl.ds(row0, ACC_ROWS_PER_SUBCORE)])


_edge_pass = pl.kernel(
    _edge_pass_kernel,
    out_type=jax.ShapeDtypeStruct((NC, ACC_ROWS, D_HID), _f32),
    mesh=_mesh,
    scratch_types=(
        [pltpu.VMEM_SHARED((ACC_ROWS, D_HID), _f32),
         pltpu.VMEM_SHARED((ACC_ROWS, D_HID), _f32),
         pltpu.VMEM((K_CHUNKS, CHUNK), jnp.int32),
         pltpu.VMEM((K_CHUNKS, CHUNK), jnp.int32)]
        + [pltpu.VMEM((CHUNK, D_HID), _f32)] * NBUF
        + [pltpu.SemaphoreType.DMA] * (2 * NBUF)
    ),
    compiler_params=pltpu.CompilerParams(use_tc_tiling_on_sc=False),
)


def _deg_pass_kernel(dst_hbm, zeros_hbm, ones_hbm, out_hbm,
                     acc_sh, dst_v, ones_v, sem):
    c = lax.axis_index("c")
    s = lax.axis_index("s")
    w = c * NS + s
    row0 = s * ACC_ROWS_PER_SUBCORE
    pltpu.sync_copy(zeros_hbm.at[pl.ds(row0, ACC_ROWS_PER_SUBCORE)],
                    acc_sh.at[pl.ds(row0, ACC_ROWS_PER_SUBCORE)])
    pltpu.sync_copy(dst_hbm.at[w], dst_v)
    pltpu.sync_copy(ones_hbm, ones_v)
    plsc.subcore_barrier()

    @pl.loop(0, K_CHUNKS // 8)
    def _(g):
        base = g * 8
        for b in range(8):
            pltpu.async_copy(ones_v, acc_sh.at[dst_v.at[base + b]], sem,
                             add=True)
        for b in range(8):
            pltpu.make_async_copy(ones_v, acc_sh.at[dst_v.at[base + b]],
                                  sem).wait()

    plsc.subcore_barrier()
    pltpu.sync_copy(acc_sh.at[pl.ds(row0, ACC_ROWS_PER_SUBCORE)],
                    out_hbm.at[c, pl.ds(row0, ACC_ROWS_PER_SUBCORE)])


_deg_pass = pl.kernel(
    _deg_pass_kernel,
    out_type=jax.ShapeDtypeStruct((NC, ACC_ROWS, D_HID), _f32),
    mesh=_mesh,
    scratch_types=[
        pltpu.VMEM_SHARED((ACC_ROWS, D_HID), _f32),
        pltpu.VMEM((K_CHUNKS, CHUNK), jnp.int32),
        pltpu.VMEM((CHUNK, D_HID), _f32),
        pltpu.SemaphoreType.DMA,
    ],
    compiler_params=pltpu.CompilerParams(use_tc_tiling_on_sc=False),
)


def _mm1_body(xf_ref, w1bd_ref, hf_ref):
    hf_ref[...] = jnp.dot(xf_ref[...], w1bd_ref[...],
                          preferred_element_type=_f32,
                          precision=lax.Precision.HIGHEST)


def _prep2_body(degpf_ref, h1f_ref, dinvf_ref, g1f_ref):
    deg = degpf_ref[0] + degpf_ref[1] + 1.0
    dinvf = lax.rsqrt(deg)
    dinvf_ref[...] = dinvf
    g1f_ref[...] = jnp.concatenate(
        [dinvf[:FROWS_REAL] * h1f_ref[...],
         jnp.zeros((FROWS - FROWS_REAL, 128), _f32)], axis=0)


def _mid_body(accpf_ref, g1f_ref, dinvf_ref, b1f_ref, w2bd_ref, g2f_ref):
    agg = accpf_ref[0] + accpf_ref[1] + g1f_ref[...]
    h = jnp.maximum(dinvf_ref[...] * agg + b1f_ref[...], 0.0)
    h2 = jnp.dot(h, w2bd_ref[...], preferred_element_type=_f32,
                 precision=lax.Precision.HIGHEST)
    g2f_ref[...] = dinvf_ref[...] * h2


def _final_body(accpf_ref, g2f_ref, dinvf_ref, b2f_ref, of_ref):
    agg = accpf_ref[0, :FROWS_REAL] + accpf_ref[1, :FROWS_REAL] \
        + g2f_ref[0:FROWS_REAL]
    of_ref[...] = dinvf_ref[0:FROWS_REAL] * agg + b2f_ref[...]


def kernel(x, edge_index, W1, b1, W2, b2):
    flat = edge_index.astype(jnp.int32).reshape(2 * N_EDGES)
    both = jnp.concatenate(
        [flat[:N_EDGES], jnp.zeros((E_PAD - N_EDGES,), jnp.int32),
         flat[N_EDGES:], jnp.full((E_PAD - N_EDGES,), N_NODES, jnp.int32)])
    src = both[:E_PAD].reshape(NW, K_CHUNKS, CHUNK)
    dst = both[E_PAD:].reshape(NW, K_CHUNKS, CHUNK)
    zeros = jnp.zeros((ACC_ROWS, D_HID), _f32)
    ones = jnp.ones((CHUNK, D_HID), _f32)
    eye8 = jnp.eye(8, dtype=_f32)
    w1bd = jnp.kron(eye8, W1)
    W2p = jnp.pad(W2, ((0, 0), (0, D_HID - N_CLASSES)))
    w2bd = jnp.kron(eye8, W2p)
    b1f = jnp.tile(b1, 8).reshape(1, 128)
    b2f = jnp.tile(jnp.pad(b2, (0, D_HID - N_CLASSES)), 8).reshape(1, 128)
    xf = x.reshape(FROWS_REAL, XF_COLS)

    degp = _deg_pass(dst, zeros, ones)
    degpf = degp.reshape(NC, FROWS, 128)
    h1f = pl.pallas_call(
        _mm1_body,
        out_shape=jax.ShapeDtypeStruct((FROWS_REAL, 128), _f32),
    )(xf, w1bd)
    dinvf, g1f = pl.pallas_call(
        _prep2_body,
        out_shape=(jax.ShapeDtypeStruct((FROWS, 128), _f32),
                   jax.ShapeDtypeStruct((FROWS, 128), _f32)),
    )(degpf, h1f)
    acc1 = _edge_pass(g1f.reshape(ACC_ROWS, D_HID), src, dst, zeros)
    g2f = pl.pallas_call(
        _mid_body,
        out_shape=jax.ShapeDtypeStruct((FROWS, 128), _f32),
    )(acc1.reshape(NC, FROWS, 128), g1f, dinvf, b1f, w2bd)
    acc2 = _edge_pass(g2f.reshape(ACC_ROWS, D_HID), src, dst, zeros)
    resf = pl.pallas_call(
        _final_body,
        out_shape=jax.ShapeDtypeStruct((FROWS_REAL, 128), _f32),
    )(acc2.reshape(NC, FROWS, 128), g2f, dinvf, b2f)
    return resf.reshape(N_NODES, D_HID)[:, :N_CLASSES]


# speedup vs baseline: 89.1873x; 1.0713x over previous
import jax
import jax.numpy as jnp
from jax import lax
from jax.experimental import pallas as pl
from jax.experimental.pallas import tpu as pltpu
from jax.experimental.pallas import tpu_sc as plsc

N_NODES = 10000
N_EDGES = 320000
D_FEAT = 128
D_HID = 16
N_CLASSES = 10

NC, NS = 2, 16
NW = NC * NS
CHUNK = 128
NBUF = 8
GAHEAD = 4
K_CHUNKS = 80
E_PAD = NW * CHUNK * K_CHUNKS

ACC_ROWS_PER_SUBCORE = 632
ACC_ROWS = NS * ACC_ROWS_PER_SUBCORE
FROWS = ACC_ROWS * D_HID // 128
FROWS_REAL = N_NODES * D_HID // 128
XF_COLS = 8 * D_FEAT

_mesh = plsc.VectorSubcoreMesh(core_axis_name="c", subcore_axis_name="s")
_f32 = jnp.float32


def _edge_pass_kernel(g_hbm, src_hbm, dst_hbm, zeros_hbm, out_hbm,
                      acc_sh, g_sh, src_v, dst_v,
                      r0, r1, r2, r3, r4, r5, r6, r7,
                      g0, g1, g2, g3, g4, g5, g6, g7,
                      t0, t1, t2, t3, t4, t5, t6, t7):
    rows = (r0, r1, r2, r3, r4, r5, r6, r7)
    gsems = (g0, g1, g2, g3, g4, g5, g6, g7)
    ssems = (t0, t1, t2, t3, t4, t5, t6, t7)
    c = lax.axis_index("c")
    s = lax.axis_index("s")
    w = c * NS + s
    row0 = s * ACC_ROWS_PER_SUBCORE
    pltpu.sync_copy(zeros_hbm.at[pl.ds(row0, ACC_ROWS_PER_SUBCORE)],
                    acc_sh.at[pl.ds(row0, ACC_ROWS_PER_SUBCORE)])
    pltpu.sync_copy(g_hbm.at[pl.ds(row0, ACC_ROWS_PER_SUBCORE)],
                    g_sh.at[pl.ds(row0, ACC_ROWS_PER_SUBCORE)])
    pltpu.sync_copy(src_hbm.at[w], src_v)
    pltpu.sync_copy(dst_hbm.at[w], dst_v)
    plsc.subcore_barrier()

    def gather_start(j, b):
        pltpu.async_copy(g_sh.at[src_v.at[j]], rows[b], gsems[b])

    def gather_wait(j, b):
        pltpu.make_async_copy(g_sh.at[src_v.at[j]], rows[b], gsems[b]).wait()

    def scatter_start(j, b):
        pltpu.async_copy(rows[b], acc_sh.at[dst_v.at[j]], ssems[b], add=True)

    def scatter_wait(j, b):
        pltpu.make_async_copy(rows[b], acc_sh.at[dst_v.at[j]],
                              ssems[b]).wait()

    for b in range(GAHEAD):
        gather_start(b, b)
    for b in range(NBUF):
        j = b
        gather_wait(j, b)
        scatter_start(j, b)
        if j >= GAHEAD:
            scatter_wait(j - GAHEAD, (j - GAHEAD) % NBUF)
        gather_start(j + GAHEAD, (j + GAHEAD) % NBUF)

    @pl.loop(1, K_CHUNKS // NBUF - 1)
    def _(g):
        base = g * NBUF
        for b in range(NBUF):
            j = base + b
            gather_wait(j, b)
            scatter_start(j, b)
            scatter_wait(j - GAHEAD, (b - GAHEAD) % NBUF)
            gather_start(j + GAHEAD, (b + GAHEAD) % NBUF)

    tail = K_CHUNKS - NBUF
    for b in range(NBUF):
        j = tail + b
        gather_wait(j, b)
        scatter_start(j, b)
        scatter_wait(j - GAHEAD, (j - GAHEAD) % NBUF)
        if j + GAHEAD < K_CHUNKS:
            gather_start(j + GAHEAD, (j + GAHEAD) % NBUF)
    for b in range(GAHEAD):
        j = K_CHUNKS - GAHEAD + b
        scatter_wait(j, j % NBUF)

    plsc.subcore_barrier()
    pltpu.sync_copy(acc_sh.at[pl.ds(row0, ACC_ROWS_PER_SUBCORE)],
                    out_hbm.at[c, pl.ds(row0, ACC_ROWS_PER_SUBCORE)])


_edge_pass = pl.kernel(
    _edge_pass_kernel,
    out_type=jax.ShapeDtypeStruct((NC, ACC_ROWS, D_HID), _f32),
    mesh=_mesh,
    scratch_types=(
        [pltpu.VMEM_SHARED((ACC_ROWS, D_HID), _f32),
         pltpu.VMEM_SHARED((ACC_ROWS, D_HID), _f32),
         pltpu.VMEM((K_CHUNKS, CHUNK), jnp.int32),
         pltpu.VMEM((K_CHUNKS, CHUNK), jnp.int32)]
        + [pltpu.VMEM((CHUNK, D_HID), _f32)] * NBUF
        + [pltpu.SemaphoreType.DMA] * (2 * NBUF)
    ),
    compiler_params=pltpu.CompilerParams(use_tc_tiling_on_sc=False),
)


def _deg_pass_kernel(dst_hbm, zeros_hbm, ones_hbm, out_hbm,
                     acc_sh, dst_v, ones_v, sem):
    c = lax.axis_index("c")
    s = lax.axis_index("s")
    w = c * NS + s
    row0 = s * ACC_ROWS_PER_SUBCORE
    pltpu.sync_copy(zeros_hbm.at[pl.ds(row0, ACC_ROWS_PER_SUBCORE)],
                    acc_sh.at[pl.ds(row0, ACC_ROWS_PER_SUBCORE)])
    pltpu.sync_copy(dst_hbm.at[w], dst_v)
    pltpu.sync_copy(ones_hbm, ones_v)
    plsc.subcore_barrier()

    @pl.loop(0, K_CHUNKS // 8)
    def _(g):
        base = g * 8
        for b in range(8):
            pltpu.async_copy(ones_v, acc_sh.at[dst_v.at[base + b]], sem,
                             add=True)
        for b in range(8):
            pltpu.make_async_copy(ones_v, acc_sh.at[dst_v.at[base + b]],
                                  sem).wait()

    plsc.subcore_barrier()
    pltpu.sync_copy(acc_sh.at[pl.ds(row0, ACC_ROWS_PER_SUBCORE)],
                    out_hbm.at[c, pl.ds(row0, ACC_ROWS_PER_SUBCORE)])


_deg_pass = pl.kernel(
    _deg_pass_kernel,
    out_type=jax.ShapeDtypeStruct((NC, ACC_ROWS, D_HID), _f32),
    mesh=_mesh,
    scratch_types=[
        pltpu.VMEM_SHARED((ACC_ROWS, D_HID), _f32),
        pltpu.VMEM((K_CHUNKS, CHUNK), jnp.int32),
        pltpu.VMEM((CHUNK, D_HID), _f32),
        pltpu.SemaphoreType.DMA,
    ],
    compiler_params=pltpu.CompilerParams(use_tc_tiling_on_sc=False),
)


def _mm1_body(xf_ref, w1bd_ref, hf_ref):
    hf_ref[...] = jnp.dot(xf_ref[...], w1bd_ref[...],
                          preferred_element_type=_f32)


def _prep2_body(degpf_ref, h1f_ref, dinvf_ref, g1f_ref):
    deg = degpf_ref[0] + degpf_ref[1] + 1.0
    dinvf = lax.rsqrt(deg)
    dinvf_ref[...] = dinvf
    g1f_ref[...] = jnp.concatenate(
        [dinvf[:FROWS_REAL] * h1f_ref[...],
         jnp.zeros((FROWS - FROWS_REAL, 128), _f32)], axis=0)


def _mid_body(accpf_ref, g1f_ref, dinvf_ref, b1f_ref, w2bd_ref, g2f_ref):
    agg = accpf_ref[0] + accpf_ref[1] + g1f_ref[...]
    h = jnp.maximum(dinvf_ref[...] * agg + b1f_ref[...], 0.0)
    h2 = jnp.dot(h, w2bd_ref[...], preferred_element_type=_f32,
                 precision=lax.Precision.HIGHEST)
    g2f_ref[...] = dinvf_ref[...] * h2


def _final_body(accpf_ref, g2f_ref, dinvf_ref, b2f_ref, of_ref):
    agg = accpf_ref[0, :FROWS_REAL] + accpf_ref[1, :FROWS_REAL] \
        + g2f_ref[0:FROWS_REAL]
    of_ref[...] = dinvf_ref[0:FROWS_REAL] * agg + b2f_ref[...]


def kernel(x, edge_index, W1, b1, W2, b2):
    flat = edge_index.astype(jnp.int32).reshape(2 * N_EDGES)
    flat = lax.optimization_barrier(flat)
    both = jnp.concatenate(
        [flat[:N_EDGES], jnp.zeros((E_PAD - N_EDGES,), jnp.int32),
         flat[N_EDGES:], jnp.full((E_PAD - N_EDGES,), N_NODES, jnp.int32)])
    src = both[:E_PAD].reshape(NW, K_CHUNKS, CHUNK)
    dst = both[E_PAD:].reshape(NW, K_CHUNKS, CHUNK)
    zeros = jnp.zeros((ACC_ROWS, D_HID), _f32)
    ones = jnp.ones((CHUNK, D_HID), _f32)
    eye8 = jnp.eye(8, dtype=_f32)
    w1bd = jnp.kron(eye8, W1)
    W2p = jnp.pad(W2, ((0, 0), (0, D_HID - N_CLASSES)))
    w2bd = jnp.kron(eye8, W2p)
    b1f = jnp.tile(b1, 8).reshape(1, 128)
    b2f = jnp.tile(jnp.pad(b2, (0, D_HID - N_CLASSES)), 8).reshape(1, 128)
    xf = x.reshape(FROWS_REAL, XF_COLS)

    degp = _deg_pass(dst, zeros, ones)
    degpf = degp.reshape(NC, FROWS, 128)
    h1f = pl.pallas_call(
        _mm1_body,
        out_shape=jax.ShapeDtypeStruct((FROWS_REAL, 128), _f32),
    )(xf, w1bd)
    dinvf, g1f = pl.pallas_call(
        _prep2_body,
        out_shape=(jax.ShapeDtypeStruct((FROWS, 128), _f32),
                   jax.ShapeDtypeStruct((FROWS, 128), _f32)),
    )(degpf, h1f)
    acc1 = _edge_pass(g1f.reshape(ACC_ROWS, D_HID), src, dst, zeros)
    g2f = pl.pallas_call(
        _mid_body,
        out_shape=jax.ShapeDtypeStruct((FROWS, 128), _f32),
    )(acc1.reshape(NC, FROWS, 128), g1f, dinvf, b1f, w2bd)
    acc2 = _edge_pass(g2f.reshape(ACC_ROWS, D_HID), src, dst, zeros)
    resf = pl.pallas_call(
        _final_body,
        out_shape=jax.ShapeDtypeStruct((FROWS_REAL, 128), _f32),
    )(acc2.reshape(NC, FROWS, 128), g2f, dinvf, b2f)
    return resf.reshape(N_NODES, D_HID)[:, :N_CLASSES]
